# Initial kernel scaffold; baseline (speedup 1.0000x reference)
import jax
import jax.numpy as jnp
from jax.experimental import pallas as pl


def kernel(x, params):
    raise NotImplementedError("write your pallas kernel here")


# speedup vs baseline: 1.0360x; 1.0360x over previous
import functools

import jax
import jax.numpy as jnp
from jax import lax
from jax.experimental import pallas as pl

_CDIMS = (((1,), (1,)), ((), ()))


def _ball_query(xyz, new_xyz, radius, nsample):
    aa = jnp.sum(new_xyz * new_xyz, axis=-1, keepdims=True)
    bb = jnp.sum(xyz * xyz, axis=-1)[:, None, :]
    ab = jnp.einsum('bsc,bnc->bsn', new_xyz, xyz)
    sqr = aa + bb - 2.0 * ab
    valid = sqr <= radius * radius
    vf = valid.astype(jnp.float32)
    rank = jnp.cumsum(vf, axis=-1) - vf
    n = xyz.shape[1]
    n_iota = jnp.arange(n, dtype=jnp.int32)
    cols = []
    for k in range(nsample):
        sel = valid & (rank == k)
        cols.append(jnp.sum(jnp.where(sel, n_iota, 0), axis=-1))
    idx = jnp.stack(cols, axis=-1)
    cnt = jnp.sum(vf, axis=-1)
    karr = jnp.arange(nsample, dtype=jnp.float32)
    idx = jnp.where(karr[None, None, :] >= cnt[:, :, None], idx[:, :, :1], idx)
    return idx


def _gather(points, idx):
    return jax.vmap(lambda p, i: p[i])(points, idx)


def _conv_body(x_ref, w1, s1, t1, w2, s2, t2, w3, s3, t3, o_ref, *, K):
    h = x_ref[0]
    for w, s, t in ((w1, s1, t1), (w2, s2, t2), (w3, s3, t3)):
        h = lax.dot_general(h, w[...], _CDIMS)
        h = jnp.maximum(h * s[...] + t[...], 0.0)
    r, o = h.shape
    o_ref[0] = jnp.max(h.reshape(r // K, K, o), axis=1)


def _conv_call(G, Ws, gs, bs, K, st):
    B, R, C = G.shape
    rt = st * K
    grid = (B, R // rt)
    aff = []
    for g, b in zip(gs, bs):
        aff.append(g.reshape(1, -1))
        aff.append(b.reshape(1, -1))
    cout = Ws[2].shape[0]
    in_specs = [pl.BlockSpec((1, rt, C), lambda b, s: (b, s, 0))]
    args = []
    for w, g, b in zip(Ws, gs, bs):
        for a in (w, g.reshape(1, -1), b.reshape(1, -1)):
            args.append(a)
            in_specs.append(pl.BlockSpec(a.shape, lambda b, s: tuple(0 for _ in a.shape)))
    return pl.pallas_call(
        functools.partial(_conv_body, K=K),
        grid=grid,
        in_specs=in_specs,
        out_specs=pl.BlockSpec((1, st, cout), lambda b, s: (b, s, 0)),
        out_shape=jax.ShapeDtypeStruct((B, R // K, cout), jnp.float32),
    )(G, *args)


def _tail_body(x_ref, w1, s1, t1, w2, s2, t2, w3, s3, t3,
               l1, u1, v1, l2, u2, v2, fw, fb, o_ref):
    h = x_ref[0]
    for w, s, t in ((w1, s1, t1), (w2, s2, t2), (w3, s3, t3)):
        h = lax.dot_general(h, w[...], _CDIMS)
        h = jnp.maximum(h * s[...] + t[...], 0.0)
    z = jnp.max(h, axis=0, keepdims=True)
    z = jnp.maximum(lax.dot_general(z, l1[...], _CDIMS) * u1[...] + v1[...], 0.0)
    z = jnp.maximum(lax.dot_general(z, l2[...], _CDIMS) * u2[...] + v2[...], 0.0)
    o_ref[0] = lax.dot_general(z, fw[...], _CDIMS) + fb[...]


def _tail_call(X3, params):
    B = X3.shape[0]
    args = []
    for w, g, b in zip(params['conv3_W'], params['conv3_g'], params['conv3_b']):
        args += [w, g.reshape(1, -1), b.reshape(1, -1)]
    args += [params['lin1_W'], params['bn1_g'].reshape(1, -1), params['bn1_b'].reshape(1, -1)]
    args += [params['lin2_W'], params['bn2_g'].reshape(1, -1), params['bn2_b'].reshape(1, -1)]
    args += [params['fc_W'], params['fc_b'].reshape(1, -1)]
    in_specs = [pl.BlockSpec((1,) + X3.shape[1:], lambda b: (b, 0, 0))]
    for a in args:
        in_specs.append(pl.BlockSpec(a.shape, lambda b: tuple(0 for _ in a.shape)))
    out = pl.pallas_call(
        _tail_body,
        grid=(B,),
        in_specs=in_specs,
        out_specs=pl.BlockSpec((1, 1, 40), lambda b: (b, 0, 0)),
        out_shape=jax.ShapeDtypeStruct((B, 1, 40), jnp.float32),
    )(X3, *args)
    return out.reshape(B, 40)


def kernel(x, params):
    xyz = jnp.transpose(x, (0, 2, 1))
    B, N, _ = xyz.shape

    S1, K1 = N // 2, 32
    c1 = xyz[:, :S1]
    gidx1 = _ball_query(xyz, c1, 0.2, K1)
    g1 = _gather(xyz, gidx1) - c1[:, :, None, :]
    G1 = g1.reshape(B, S1 * K1, 3)
    F1 = _conv_call(G1, params['conv1_W'], params['conv1_g'], params['conv1_b'], K1, 64)

    S2, K2 = S1 // 2, 64
    c2 = c1[:, :S2]
    gidx2 = _ball_query(c1, c2, 0.4, K2)
    gx = _gather(c1, gidx2) - c2[:, :, None, :]
    gf = _gather(F1, gidx2)
    G2 = jnp.concatenate([gx, gf], axis=-1).reshape(B, S2 * K2, 131)
    F2 = _conv_call(G2, params['conv2_W'], params['conv2_g'], params['conv2_b'], K2, 32)

    X3 = jnp.concatenate([c2, F2], axis=-1)
    return _tail_call(X3, params)


# speedup vs baseline: 15.8954x; 15.3427x over previous
import functools

import jax
import jax.numpy as jnp
from jax import lax
from jax.experimental import pallas as pl
from jax.experimental.pallas import tpu as pltpu
from jax.experimental.pallas import tpu_sc as plsc

_CDIMS = (((1,), (1,)), ((), ()))


def _bdot(h, w):
    return lax.dot_general(h.astype(jnp.bfloat16), w, _CDIMS,
                           preferred_element_type=jnp.float32)


def _sqdist(new_xyz, xyz):
    aa = jnp.sum(new_xyz * new_xyz, axis=-1, keepdims=True)
    bb = jnp.sum(xyz * xyz, axis=-1)[:, None, :]
    ab = jnp.einsum('bsc,bnc->bsn', new_xyz, xyz)
    return aa + bb - 2.0 * ab


def _bq_body(sqr_ref, gidx_ref, *, K, r2, NB):
    S, N = sqr_ref.shape[1:]
    valid = sqr_ref[0] <= r2
    vf = valid.astype(jnp.float32)
    tri = (lax.broadcasted_iota(jnp.int32, (N, N), 0)
           < lax.broadcasted_iota(jnp.int32, (N, N), 1))
    rank = lax.dot_general(vf.astype(jnp.bfloat16), tri.astype(jnp.bfloat16),
                           (((1,), (0,)), ((), ())),
                           preferred_element_type=jnp.float32)
    cnt = jnp.sum(vf, axis=1, keepdims=True)
    n_iota = lax.broadcasted_iota(jnp.int32, (1, N), 1).astype(jnp.float32)
    karr = lax.broadcasted_iota(jnp.int32, (1, K), 1).astype(jnp.float32)
    idx = jnp.zeros((S, K), jnp.float32)
    col0 = None
    for k in range(K):
        sel = valid & (rank == float(k))
        colk = jnp.sum(jnp.where(sel, n_iota, 0.0), axis=1, keepdims=True)
        if k == 0:
            col0 = colk
        idx = idx + colk * (karr == float(k)).astype(jnp.float32)
    idx = jnp.where(karr >= cnt, col0, idx)
    boff = (pl.program_id(0) * NB).astype(jnp.float32)
    gidx_ref[0] = (idx + boff).astype(jnp.int32)


def _bq_call(xt, S, K, r2):
    B, N, _ = xt.shape
    sqr = _sqdist(xt[:, :S], xt)
    return pl.pallas_call(
        functools.partial(_bq_body, K=K, r2=r2, NB=N),
        grid=(B,),
        in_specs=[pl.BlockSpec((1, S, N), lambda b: (b, 0, 0))],
        out_specs=pl.BlockSpec((1, S, K), lambda b: (b, 0, 0)),
        out_shape=jax.ShapeDtypeStruct((B, S, K), jnp.int32),
    )(sqr)


def _sc_gather(table, idx2d):
    V, D = table.shape
    M = idx2d.shape[0] * 128
    NW = 32
    per_w = M // NW
    CH = 512 if D <= 128 else 256
    parts = 1024 // CH
    mesh = plsc.VectorSubcoreMesh(core_axis_name="c", subcore_axis_name="s")

    @functools.partial(
        pl.kernel,
        out_type=jax.ShapeDtypeStruct((M, D), jnp.float32),
        mesh=mesh,
        scratch_types=[
            pltpu.VMEM((8, 128), jnp.int32),
            pltpu.VMEM((CH, D), jnp.float32),
            pltpu.SemaphoreType.DMA,
        ],
    )
    def k(table_hbm, idx_hbm, out_hbm, idx_v, rows_v, sem):
        wid = lax.axis_index("s") * 2 + lax.axis_index("c")
        base = wid * per_w

        def chunk(i, carry):
            rbase = pl.multiple_of(base + i * 1024, 1024)
            pltpu.sync_copy(idx_hbm.at[pl.ds(pl.multiple_of(rbase // 128, 8), 8)], idx_v)
            jpp = CH // 128
            for part in range(parts):
                cps = [
                    pltpu.async_copy(table_hbm.at[idx_v.at[part * jpp + j]],
                                     rows_v.at[pl.ds(j * 128, 128)], sem)
                    for j in range(jpp)
                ]
                for cp in cps:
                    cp.wait()
                pltpu.sync_copy(rows_v,
                                out_hbm.at[pl.ds(rbase + part * CH, CH)])
            return carry

        lax.fori_loop(0, per_w // 1024, chunk, 0)

    return k(table, idx2d)


def _conv_body(gy_ref, c_ref, w1, s1, t1, w2, s2, t2, w3, s3, t3, o_ref, *, K):
    g = gy_ref[0]
    stk, c1 = g.shape
    st = stk // K
    h = (g.reshape(st, K, c1) - c_ref[0][:, None, :]).reshape(stk, c1)
    for w, s, t in ((w1, s1, t1), (w2, s2, t2), (w3, s3, t3)):
        h = _bdot(h, w[...])
        h = jnp.maximum(h * s[...] + t[...], 0.0)
    o_ref[0] = jnp.max(h.reshape(st, K, h.shape[1]), axis=1)


def _conv_call(Gy, carg, Ws, gs, bs, K, st):
    B, R, C = Gy.shape
    rt = st * K
    grid = (B, R // rt)
    cout = Ws[2].shape[0]
    in_specs = [
        pl.BlockSpec((1, rt, C), lambda b, s: (b, s, 0)),
        pl.BlockSpec((1, st, C), lambda b, s: (b, s, 0)),
    ]
    args = []
    for w, g, b in zip(Ws, gs, bs):
        for a in (w.astype(jnp.bfloat16), g.reshape(1, -1), b.reshape(1, -1)):
            args.append(a)
            in_specs.append(pl.BlockSpec(a.shape, lambda b, s: (0, 0)))
    return pl.pallas_call(
        functools.partial(_conv_body, K=K),
        grid=grid,
        in_specs=in_specs,
        out_specs=pl.BlockSpec((1, st, cout), lambda b, s: (b, s, 0)),
        out_shape=jax.ShapeDtypeStruct((B, R // K, cout), jnp.float32),
    )(Gy, carg, *args)


def _tail_body(x_ref, w1, s1, t1, w2, s2, t2, w3, s3, t3,
               l1, u1, v1, l2, u2, v2, fw, fb, o_ref):
    h = x_ref[0]
    for w, s, t in ((w1, s1, t1), (w2, s2, t2), (w3, s3, t3)):
        h = _bdot(h, w[...])
        h = jnp.maximum(h * s[...] + t[...], 0.0)
    z = jnp.max(h, axis=0, keepdims=True)
    z = jnp.maximum(_bdot(z, l1[...]) * u1[...] + v1[...], 0.0)
    z = jnp.maximum(_bdot(z, l2[...]) * u2[...] + v2[...], 0.0)
    o_ref[0] = _bdot(z, fw[...]) + fb[...]


def _tail_call(X3, params):
    B = X3.shape[0]
    args = []
    for w, g, b in zip(params['conv3_W'], params['conv3_g'], params['conv3_b']):
        args += [w.astype(jnp.bfloat16), g.reshape(1, -1), b.reshape(1, -1)]
    args += [params['lin1_W'].astype(jnp.bfloat16),
             params['bn1_g'].reshape(1, -1), params['bn1_b'].reshape(1, -1)]
    args += [params['lin2_W'].astype(jnp.bfloat16),
             params['bn2_g'].reshape(1, -1), params['bn2_b'].reshape(1, -1)]
    args += [params['fc_W'].astype(jnp.bfloat16), params['fc_b'].reshape(1, -1)]
    in_specs = [pl.BlockSpec((1,) + X3.shape[1:], lambda b: (b, 0, 0))]
    for a in args:
        in_specs.append(pl.BlockSpec(a.shape, lambda b: tuple(0 for _ in a.shape)))
    out = pl.pallas_call(
        _tail_body,
        grid=(B,),
        in_specs=in_specs,
        out_specs=pl.BlockSpec((1, 1, 40), lambda b: (b, 0, 0)),
        out_shape=jax.ShapeDtypeStruct((B, 1, 40), jnp.float32),
    )(X3, *args)
    return out.reshape(B, 40)


def kernel(x, params):
    B, _, N = x.shape
    xt = jnp.transpose(x, (0, 2, 1))

    S1, K1 = N // 2, 32
    gidx1 = _bq_call(xt, S1, K1, 0.2 * 0.2)
    X1p = jnp.pad(xt, ((0, 0), (0, 0), (0, 125)))
    w1p = jnp.pad(params['conv1_W'][0], ((0, 0), (0, 125)))
    G1 = _sc_gather(X1p.reshape(B * N, 128), gidx1.reshape(-1, 128))
    F1 = _conv_call(G1.reshape(B, S1 * K1, 128), X1p[:, :S1],
                    [w1p] + params['conv1_W'][1:], params['conv1_g'],
                    params['conv1_b'], K1, 64)

    S2, K2 = S1 // 2, 64
    xt2 = xt[:, :S1]
    gidx2 = _bq_call(xt2, S2, K2, 0.4 * 0.4)
    X2p = jnp.concatenate(
        [xt2, F1, jnp.zeros((B, S1, 125), jnp.float32)], axis=-1)
    w2p = jnp.pad(params['conv2_W'][0], ((0, 0), (0, 125)))
    c2 = jnp.pad(xt2[:, :S2], ((0, 0), (0, 0), (0, 253)))
    G2 = _sc_gather(X2p.reshape(B * S1, 256), gidx2.reshape(-1, 128))
    F2 = _conv_call(G2.reshape(B, S2 * K2, 256), c2,
                    [w2p] + params['conv2_W'][1:], params['conv2_g'],
                    params['conv2_b'], K2, 32)

    X3 = jnp.concatenate([xt2[:, :S2], F2], axis=-1)
    return _tail_call(X3, params)


# speedup vs baseline: 17.3856x; 1.0938x over previous
import functools

import jax
import jax.numpy as jnp
from jax import lax
from jax.experimental import pallas as pl
from jax.experimental.pallas import tpu as pltpu
from jax.experimental.pallas import tpu_sc as plsc

_CDIMS = (((1,), (1,)), ((), ()))


def _bdot(h, w):
    return lax.dot_general(h.astype(jnp.bfloat16), w, _CDIMS,
                           preferred_element_type=jnp.float32)


def _sqdist(new_xyz, xyz):
    aa = jnp.sum(new_xyz * new_xyz, axis=-1, keepdims=True)
    bb = jnp.sum(xyz * xyz, axis=-1)[:, None, :]
    ab = jnp.einsum('bsc,bnc->bsn', new_xyz, xyz)
    return aa + bb - 2.0 * ab


def _bq_body(sqr_ref, gidx_ref, *, K, r2, NB):
    S, N = sqr_ref.shape[1:]
    valid = sqr_ref[0] <= r2
    vf = valid.astype(jnp.float32)
    tri = (lax.broadcasted_iota(jnp.int32, (N, N), 0)
           < lax.broadcasted_iota(jnp.int32, (N, N), 1))
    rank = lax.dot_general(vf.astype(jnp.bfloat16), tri.astype(jnp.bfloat16),
                           (((1,), (0,)), ((), ())),
                           preferred_element_type=jnp.float32)
    cnt = jnp.sum(vf, axis=1, keepdims=True)
    n_iota = lax.broadcasted_iota(jnp.int32, (1, N), 1).astype(jnp.float32)
    karr = lax.broadcasted_iota(jnp.int32, (1, K), 1).astype(jnp.float32)
    idx = jnp.zeros((S, K), jnp.float32)
    col0 = None
    for k in range(K):
        sel = valid & (rank == float(k))
        colk = jnp.sum(jnp.where(sel, n_iota, 0.0), axis=1, keepdims=True)
        if k == 0:
            col0 = colk
        idx = idx + colk * (karr == float(k)).astype(jnp.float32)
    idx = jnp.where(karr >= cnt, col0, idx)
    boff = (pl.program_id(0) * NB).astype(jnp.float32)
    gidx_ref[0] = (idx + boff).astype(jnp.int32)


def _bq_call(xt, S, K, r2):
    B, N, _ = xt.shape
    sqr = _sqdist(xt[:, :S], xt)
    return pl.pallas_call(
        functools.partial(_bq_body, K=K, r2=r2, NB=N),
        grid=(B,),
        in_specs=[pl.BlockSpec((1, S, N), lambda b: (b, 0, 0))],
        out_specs=pl.BlockSpec((1, S, K), lambda b: (b, 0, 0)),
        out_shape=jax.ShapeDtypeStruct((B, S, K), jnp.int32),
    )(sqr)


def _sc_gather(table, idx2d):
    V, D = table.shape
    M = idx2d.shape[0] * 128
    NW = 32
    per_w = M // NW
    CH = 512 if D <= 128 else 256
    parts = 1024 // CH
    mesh = plsc.VectorSubcoreMesh(core_axis_name="c", subcore_axis_name="s")

    @functools.partial(
        pl.kernel,
        out_type=jax.ShapeDtypeStruct((M, D), jnp.float32),
        mesh=mesh,
        scratch_types=[
            pltpu.VMEM((8, 128), jnp.int32),
            pltpu.VMEM((CH, D), jnp.float32),
            pltpu.SemaphoreType.DMA,
        ],
    )
    def k(table_hbm, idx_hbm, out_hbm, idx_v, rows_v, sem):
        wid = lax.axis_index("s") * 2 + lax.axis_index("c")
        base = wid * per_w

        def chunk(i, carry):
            rbase = pl.multiple_of(base + i * 1024, 1024)
            pltpu.sync_copy(idx_hbm.at[pl.ds(pl.multiple_of(rbase // 128, 8), 8)], idx_v)
            jpp = CH // 128
            for part in range(parts):
                cps = [
                    pltpu.async_copy(table_hbm.at[idx_v.at[part * jpp + j]],
                                     rows_v.at[pl.ds(j * 128, 128)], sem)
                    for j in range(jpp)
                ]
                for cp in cps:
                    cp.wait()
                pltpu.sync_copy(rows_v,
                                out_hbm.at[pl.ds(rbase + part * CH, CH)])
            return carry

        lax.fori_loop(0, per_w // 1024, chunk, 0)

    return k(table, idx2d)


def _conv_body(gy_ref, c_ref, w1, s1, t1, w2, s2, t2, w3, s3, t3, o_ref, *, K):
    g = gy_ref[0]
    stk, c1 = g.shape
    st = stk // K
    h = (g.reshape(st, K, c1) - c_ref[0][:, None, :]).reshape(stk, c1)
    for w, s, t in ((w1, s1, t1), (w2, s2, t2), (w3, s3, t3)):
        h = _bdot(h, w[...])
        h = jnp.maximum(h * s[...] + t[...], 0.0)
    o_ref[0] = jnp.max(h.reshape(st, K, h.shape[1]), axis=1)


def _conv_call(Gy, carg, Ws, gs, bs, K, st):
    B, R, C = Gy.shape
    rt = st * K
    grid = (B, R // rt)
    cout = Ws[2].shape[0]
    in_specs = [
        pl.BlockSpec((1, rt, C), lambda b, s: (b, s, 0)),
        pl.BlockSpec((1, st, C), lambda b, s: (b, s, 0)),
    ]
    args = []
    for w, g, b in zip(Ws, gs, bs):
        for a in (w.astype(jnp.bfloat16), g.reshape(1, -1), b.reshape(1, -1)):
            args.append(a)
            in_specs.append(pl.BlockSpec(a.shape, lambda b, s: (0, 0)))
    return pl.pallas_call(
        functools.partial(_conv_body, K=K),
        grid=grid,
        in_specs=in_specs,
        out_specs=pl.BlockSpec((1, st, cout), lambda b, s: (b, s, 0)),
        out_shape=jax.ShapeDtypeStruct((B, R // K, cout), jnp.float32),
    )(Gy, carg, *args)


def _tail_body(x_ref, w1, s1, t1, w2, s2, t2, w3, s3, t3,
               l1, u1, v1, l2, u2, v2, fw, fb, o_ref):
    h = x_ref[0]
    for w, s, t in ((w1, s1, t1), (w2, s2, t2), (w3, s3, t3)):
        h = _bdot(h, w[...])
        h = jnp.maximum(h * s[...] + t[...], 0.0)
    z = jnp.max(h, axis=0, keepdims=True)
    z = jnp.maximum(_bdot(z, l1[...]) * u1[...] + v1[...], 0.0)
    z = jnp.maximum(_bdot(z, l2[...]) * u2[...] + v2[...], 0.0)
    o_ref[0] = _bdot(z, fw[...]) + fb[...]


def _tail_call(X3, params):
    B = X3.shape[0]
    args = []
    for w, g, b in zip(params['conv3_W'], params['conv3_g'], params['conv3_b']):
        args += [w.astype(jnp.bfloat16), g.reshape(1, -1), b.reshape(1, -1)]
    args += [params['lin1_W'].astype(jnp.bfloat16),
             params['bn1_g'].reshape(1, -1), params['bn1_b'].reshape(1, -1)]
    args += [params['lin2_W'].astype(jnp.bfloat16),
             params['bn2_g'].reshape(1, -1), params['bn2_b'].reshape(1, -1)]
    args += [params['fc_W'].astype(jnp.bfloat16), params['fc_b'].reshape(1, -1)]
    in_specs = [pl.BlockSpec((1,) + X3.shape[1:], lambda b: (b, 0, 0))]
    for a in args:
        in_specs.append(pl.BlockSpec(a.shape, lambda b: tuple(0 for _ in a.shape)))
    out = pl.pallas_call(
        _tail_body,
        grid=(B,),
        in_specs=in_specs,
        out_specs=pl.BlockSpec((1, 1, 40), lambda b: (b, 0, 0)),
        out_shape=jax.ShapeDtypeStruct((B, 1, 40), jnp.float32),
    )(X3, *args)
    return out.reshape(B, 40)


def kernel(x, params):
    B, _, N = x.shape
    xt = jnp.transpose(x, (0, 2, 1))

    S1, K1 = N // 2, 32
    gidx1 = _bq_call(xt, S1, K1, 0.2 * 0.2)
    X1p = jnp.pad(xt, ((0, 0), (0, 0), (0, 125)))
    w1p = jnp.pad(params['conv1_W'][0], ((0, 0), (0, 125)))
    G1 = _sc_gather(X1p.reshape(B * N, 128), gidx1.reshape(-1, 128))
    F1 = _conv_call(G1.reshape(B, S1 * K1, 128), X1p[:, :S1],
                    [w1p] + params['conv1_W'][1:], params['conv1_g'],
                    params['conv1_b'], K1, 256)

    S2, K2 = S1 // 2, 64
    xt2 = xt[:, :S1]
    gidx2 = _bq_call(xt2, S2, K2, 0.4 * 0.4)
    X2p = jnp.concatenate(
        [xt2, F1, jnp.zeros((B, S1, 125), jnp.float32)], axis=-1)
    w2p = jnp.pad(params['conv2_W'][0], ((0, 0), (0, 125)))
    c2 = jnp.pad(xt2[:, :S2], ((0, 0), (0, 0), (0, 253)))
    G2 = _sc_gather(X2p.reshape(B * S1, 256), gidx2.reshape(-1, 128))
    F2 = _conv_call(G2.reshape(B, S2 * K2, 256), c2,
                    [w2p] + params['conv2_W'][1:], params['conv2_g'],
                    params['conv2_b'], K2, 64)

    X3 = jnp.concatenate([xt2[:, :S2], F2], axis=-1)
    return _tail_call(X3, params)


# speedup vs baseline: 18.3337x; 1.0545x over previous
import functools

import jax
import jax.numpy as jnp
from jax import lax
from jax.experimental import pallas as pl
from jax.experimental.pallas import tpu as pltpu
from jax.experimental.pallas import tpu_sc as plsc

_CDIMS = (((1,), (1,)), ((), ()))


def _bdot(h, w):
    return lax.dot_general(h.astype(jnp.bfloat16), w, _CDIMS,
                           preferred_element_type=jnp.float32)


def _sqdist(new_xyz, xyz):
    aa = jnp.sum(new_xyz * new_xyz, axis=-1, keepdims=True)
    bb = jnp.sum(xyz * xyz, axis=-1)[:, None, :]
    ab = jnp.einsum('bsc,bnc->bsn', new_xyz, xyz)
    return aa + bb - 2.0 * ab


def _bq_body(sqr_ref, gidx_ref, *, K, r2, NB):
    S, N = sqr_ref.shape[1:]
    valid = sqr_ref[0] <= r2
    vf = valid.astype(jnp.float32)
    tri = (lax.broadcasted_iota(jnp.int32, (N, N), 0)
           < lax.broadcasted_iota(jnp.int32, (N, N), 1))
    rank = lax.dot_general(vf.astype(jnp.bfloat16), tri.astype(jnp.bfloat16),
                           (((1,), (0,)), ((), ())),
                           preferred_element_type=jnp.float32)
    cnt = jnp.sum(vf, axis=1, keepdims=True)
    n_iota = lax.broadcasted_iota(jnp.int32, (1, N), 1).astype(jnp.float32)
    karr = lax.broadcasted_iota(jnp.int32, (1, K), 1).astype(jnp.float32)
    idx = jnp.zeros((S, K), jnp.float32)
    col0 = None
    for k in range(K):
        sel = valid & (rank == float(k))
        colk = jnp.sum(jnp.where(sel, n_iota, 0.0), axis=1, keepdims=True)
        if k == 0:
            col0 = colk
        idx = idx + colk * (karr == float(k)).astype(jnp.float32)
    idx = jnp.where(karr >= cnt, col0, idx)
    boff = (pl.program_id(0) * NB).astype(jnp.float32)
    gidx_ref[0] = (idx + boff).astype(jnp.int32)


def _bq_call(xt, S, K, r2):
    B, N, _ = xt.shape
    sqr = _sqdist(xt[:, :S], xt)
    return pl.pallas_call(
        functools.partial(_bq_body, K=K, r2=r2, NB=N),
        grid=(B,),
        in_specs=[pl.BlockSpec((1, S, N), lambda b: (b, 0, 0))],
        out_specs=pl.BlockSpec((1, S, K), lambda b: (b, 0, 0)),
        out_shape=jax.ShapeDtypeStruct((B, S, K), jnp.int32),
    )(sqr)


def _sc_gather(table, idx2d):
    V, D = table.shape
    M = idx2d.shape[0] * 128
    NW = 32
    per_w = M // NW
    CH = 1024
    mesh = plsc.VectorSubcoreMesh(core_axis_name="c", subcore_axis_name="s")

    @functools.partial(
        pl.kernel,
        out_type=jax.ShapeDtypeStruct((M, D), jnp.float32),
        mesh=mesh,
        scratch_types=[
            pltpu.VMEM((8, 128), jnp.int32),
            pltpu.VMEM((CH, D), jnp.float32),
            pltpu.SemaphoreType.DMA,
        ],
        compiler_params=pltpu.CompilerParams(use_tc_tiling_on_sc=False),
    )
    def k(table_hbm, idx_hbm, out_hbm, idx_v, rows_v, sem):
        wid = lax.axis_index("s") * 2 + lax.axis_index("c")
        base = wid * per_w

        def chunk(i, carry):
            rbase = pl.multiple_of(base + i * CH, CH)
            pltpu.sync_copy(idx_hbm.at[pl.ds(pl.multiple_of(rbase // 128, 8), 8)], idx_v)
            cps = [
                pltpu.async_copy(table_hbm.at[idx_v.at[j]],
                                 rows_v.at[pl.ds(j * 128, 128)], sem)
                for j in range(8)
            ]
            for cp in cps:
                cp.wait()
            pltpu.sync_copy(rows_v, out_hbm.at[pl.ds(rbase, CH)])
            return carry

        lax.fori_loop(0, per_w // CH, chunk, 0)

    return k(table, idx2d)


def _conv_body(gy_ref, c_ref, w1, s1, t1, w2, s2, t2, w3, s3, t3, o_ref, *,
               K, packed):
    g = gy_ref[0]
    stk, cw = g.shape
    st = stk // K
    g3 = g.reshape(st, K, cw)
    if packed:
        rel = g3[:, :, :16] - c_ref[0][:, None, :]
        u = lax.bitcast_convert_type(g3[:, :, 16:], jnp.uint32)
        lo = lax.bitcast_convert_type(u << 16, jnp.float32)
        hi = lax.bitcast_convert_type(u & jnp.uint32(0xFFFF0000), jnp.float32)
        h = jnp.concatenate([rel, lo, hi], axis=2).reshape(stk, 2 * cw - 16)
    else:
        h = (g3 - c_ref[0][:, None, :]).reshape(stk, cw)
    for w, s, t in ((w1, s1, t1), (w2, s2, t2), (w3, s3, t3)):
        h = _bdot(h, w[...])
        h = jnp.maximum(h * s[...] + t[...], 0.0)
    o_ref[0] = jnp.max(h.reshape(st, K, h.shape[1]), axis=1)


def _conv_call(Gy, carg, Ws, gs, bs, K, st, packed=False):
    B, R, C = Gy.shape
    rt = st * K
    grid = (B, R // rt)
    cout = Ws[2].shape[0]
    in_specs = [
        pl.BlockSpec((1, rt, C), lambda b, s: (b, s, 0)),
        pl.BlockSpec((1, st, carg.shape[2]), lambda b, s: (b, s, 0)),
    ]
    args = []
    for w, g, b in zip(Ws, gs, bs):
        for a in (w.astype(jnp.bfloat16), g.reshape(1, -1), b.reshape(1, -1)):
            args.append(a)
            in_specs.append(pl.BlockSpec(a.shape, lambda b, s: (0, 0)))
    return pl.pallas_call(
        functools.partial(_conv_body, K=K, packed=packed),
        grid=grid,
        in_specs=in_specs,
        out_specs=pl.BlockSpec((1, st, cout), lambda b, s: (b, s, 0)),
        out_shape=jax.ShapeDtypeStruct((B, R // K, cout), jnp.float32),
    )(Gy, carg, *args)


def _tail_body(x_ref, w1, s1, t1, w2, s2, t2, w3, s3, t3,
               l1, u1, v1, l2, u2, v2, fw, fb, o_ref):
    h = x_ref[0]
    for w, s, t in ((w1, s1, t1), (w2, s2, t2), (w3, s3, t3)):
        h = _bdot(h, w[...])
        h = jnp.maximum(h * s[...] + t[...], 0.0)
    z = jnp.max(h, axis=0, keepdims=True)
    z = jnp.maximum(_bdot(z, l1[...]) * u1[...] + v1[...], 0.0)
    z = jnp.maximum(_bdot(z, l2[...]) * u2[...] + v2[...], 0.0)
    o_ref[0] = _bdot(z, fw[...]) + fb[...]


def _tail_call(X3, params):
    B = X3.shape[0]
    args = []
    for w, g, b in zip(params['conv3_W'], params['conv3_g'], params['conv3_b']):
        args += [w.astype(jnp.bfloat16), g.reshape(1, -1), b.reshape(1, -1)]
    args += [params['lin1_W'].astype(jnp.bfloat16),
             params['bn1_g'].reshape(1, -1), params['bn1_b'].reshape(1, -1)]
    args += [params['lin2_W'].astype(jnp.bfloat16),
             params['bn2_g'].reshape(1, -1), params['bn2_b'].reshape(1, -1)]
    args += [params['fc_W'].astype(jnp.bfloat16), params['fc_b'].reshape(1, -1)]
    in_specs = [pl.BlockSpec((1,) + X3.shape[1:], lambda b: (b, 0, 0))]
    for a in args:
        in_specs.append(pl.BlockSpec(a.shape, lambda b: tuple(0 for _ in a.shape)))
    out = pl.pallas_call(
        _tail_body,
        grid=(B,),
        in_specs=in_specs,
        out_specs=pl.BlockSpec((1, 1, 40), lambda b: (b, 0, 0)),
        out_shape=jax.ShapeDtypeStruct((B, 1, 40), jnp.float32),
    )(X3, *args)
    return out.reshape(B, 40)


def kernel(x, params):
    B, _, N = x.shape
    xt = jnp.transpose(x, (0, 2, 1))

    S1, K1 = N // 2, 32
    gidx1 = _bq_call(xt, S1, K1, 0.2 * 0.2)
    X1p = jnp.pad(xt, ((0, 0), (0, 0), (0, 13)))
    w1p = jnp.pad(params['conv1_W'][0], ((0, 0), (0, 13)))
    G1 = _sc_gather(X1p.reshape(B * N, 16), gidx1.reshape(-1, 128))
    F1 = _conv_call(G1.reshape(B, S1 * K1, 16), X1p[:, :S1],
                    [w1p] + params['conv1_W'][1:], params['conv1_g'],
                    params['conv1_b'], K1, 256)

    S2, K2 = S1 // 2, 64
    xt2 = xt[:, :S1]
    gidx2 = _bq_call(xt2, S2, K2, 0.4 * 0.4)
    F1b = F1.astype(jnp.bfloat16)
    packF1 = lax.bitcast_convert_type(
        jnp.stack([F1b[:, :, :64], F1b[:, :, 64:]], axis=-1), jnp.float32)
    X2p = jnp.concatenate(
        [xt2, jnp.zeros((B, S1, 13), jnp.float32), packF1], axis=-1)
    w21 = params['conv2_W'][0]
    w2p = jnp.concatenate(
        [w21[:, :3], jnp.zeros((128, 13), jnp.float32), w21[:, 3:]], axis=1)
    c2 = jnp.pad(xt2[:, :S2], ((0, 0), (0, 0), (0, 13)))
    G2 = _sc_gather(X2p.reshape(B * S1, 80), gidx2.reshape(-1, 128))
    F2 = _conv_call(G2.reshape(B, S2 * K2, 80), c2,
                    [w2p] + params['conv2_W'][1:], params['conv2_g'],
                    params['conv2_b'], K2, 64, packed=True)

    X3 = jnp.concatenate([xt2[:, :S2], F2], axis=-1)
    return _tail_call(X3, params)
